# baseline (speedup 1.0000x reference)
import jax
import jax.numpy as jnp
from jax import lax
from jax.experimental import pallas as pl
from jax.experimental.pallas import tpu as pltpu
from jax.experimental.pallas import tpu_sc as plsc

_NEG_RATIO = 3.0

_NC = 2
_NS = 16
_LANE = 16
_NW = _NC * _NS

_NB = 1024
_TBL = _NB * _LANE
_CSH = 21
_FSH = 11

_B = 16
_N = _B * 512 * 512
_CH = 8192
_ROWS = _CH // 512
_NCH = (_N // _NW) // _CH
_UNROLL = 8


_SL = _TBL // _NS


def _tile_merge(cnt_a, cnt_b, shared, tmp_v, merged_v, out_hbm):
    sid = lax.axis_index("s")
    cid = lax.axis_index("c")

    @plsc.parallel_loop(0, _TBL // _LANE, unroll=8)
    def _bmerge(i):
        s = pl.ds(i * _LANE, _LANE)
        cnt_a[s] = cnt_a[s] + cnt_b[s]

    half = _TBL // 2
    for h in range(2):
        pltpu.sync_copy(cnt_a.at[pl.ds(h * half, half)], shared.at[sid])
        plsc.subcore_barrier()

        @pl.when(lax.shift_right_logical(sid, 3) == h)
        def _reduce_slice():
            pltpu.sync_copy(
                shared.at[:, pl.ds(sid * _SL - h * half, _SL)], tmp_v)

            @plsc.parallel_loop(0, _SL // _LANE, unroll=4)
            def _acc(i):
                s = pl.ds(i * _LANE, _LANE)
                tot = tmp_v[0, s]
                for r in range(1, _NS):
                    tot = tot + tmp_v[r, s]
                merged_v[s] = tot

            pltpu.sync_copy(merged_v, out_hbm.at[cid, pl.ds(sid * _SL, _SL)])

        plsc.subcore_barrier()


def _coarse_body(pred_hbm, gt_hbm, mask_hbm, cnt_hbm, neg_hbm, stat_hbm,
                 bp0, bp1, bg0, bg1, bm0, bm1, bn0, bn1, stat_v,
                 cnt_a, cnt_b, shared, tmp_v, merged_v,
                 semp0, semp1, semg0, semg1, semm0, semm1, semw0, semw1):
    wid = lax.axis_index("s") * _NC + lax.axis_index("c")

    @plsc.parallel_loop(0, _TBL // _LANE, unroll=8)
    def _zero(i):
        cnt_a[pl.ds(i * _LANE, _LANE)] = jnp.zeros((_LANE,), jnp.float32)
        cnt_b[pl.ds(i * _LANE, _LANE)] = jnp.zeros((_LANE,), jnp.float32)

    lane = lax.iota(jnp.int32, _LANE)
    ones = jnp.ones((_LANE,), jnp.float32)
    fone = jnp.ones((_LANE,), jnp.float32)
    fzero = jnp.zeros((_LANE,), jnp.float32)
    shift = jnp.full((_LANE,), _CSH, jnp.int32)
    sixteen = jnp.full((_LANE,), _LANE, jnp.int32)

    b_idx = lax.shift_right_logical(wid, 1)
    r_base = (wid & 1) * 256
    bps = (bp0, bp1)
    bgs = (bg0, bg1)
    bms = (bm0, bm1)
    bns = (bn0, bn1)
    semws = (semw0, semw1)

    def _reads(c):
        rows = pl.ds(r_base + c * _ROWS, _ROWS)
        i = c % 2
        return (pltpu.async_copy(pred_hbm.at[b_idx, 0, rows, :], bps[i],
                                 (semp0, semp1)[i]),
                pltpu.async_copy(gt_hbm.at[b_idx, rows, :], bgs[i],
                                 (semg0, semg1)[i]),
                pltpu.async_copy(mask_hbm.at[b_idx, rows, :], bms[i],
                                 (semm0, semm1)[i]))

    def _chunk(i, carry):
        bp, bg, bm, bn = bps[i], bgs[i], bms[i], bns[i]

        @plsc.parallel_loop(0, _CH // _LANE, step=2, unroll=_UNROLL // 2,
                            carry=carry)
        def _vreg(j, carry):
            psum, pcnt = carry
            for u, cnt_v in ((0, cnt_a), (1, cnt_b)):
                jj = j + u
                r = lax.shift_right_logical(jj, 5)
                cc = (jj & 31) * _LANE
                p = bp[r, pl.ds(cc, _LANE)]
                g = bg[r, pl.ds(cc, _LANE)]
                m = bm[r, pl.ds(cc, _LANE)]
                loss = jnp.abs(p - g)
                lm = loss * m
                neg = loss - lm
                bn[r, pl.ds(cc, _LANE)] = neg
                psum = psum + lm
                pcnt = pcnt + m
                bits = lax.bitcast_convert_type(neg, jnp.int32)
                fb = lax.shift_right_logical(bits, shift)
                idx = fb * sixteen + lane
                plsc.addupdate_scatter(cnt_v, [idx], ones)
            return (psum, pcnt)

        return _vreg

    carry = (fzero, fzero)
    writes = [None, None]
    cur = _reads(0)
    for c in range(_NCH):
        nxt = _reads(c + 1) if c + 1 < _NCH else None
        for d in cur:
            d.wait()
        if writes[c % 2] is not None:
            writes[c % 2].wait()
        carry = _chunk(c % 2, carry)
        writes[c % 2] = pltpu.async_copy(
            bns[c % 2],
            neg_hbm.at[b_idx, pl.ds(r_base + c * _ROWS, _ROWS), :],
            semws[c % 2])
        cur = nxt

    for w in writes:
        if w is not None:
            w.wait()

    psum, pcnt = carry
    stat_v[0, :] = psum
    stat_v[1, :] = pcnt
    pltpu.sync_copy(stat_v, stat_hbm.at[wid])
    _tile_merge(cnt_a, cnt_b, shared, tmp_v, merged_v, cnt_hbm)


def _fine_body(neg_hbm, par_hbm, cnt_hbm, sab_hbm,
               buf0, buf1, par_v, sab_v, cnt_a, cnt_b,
               shared, tmp_v, merged_v, sem0, sem1):
    wid = lax.axis_index("s") * _NC + lax.axis_index("c")

    pltpu.sync_copy(par_hbm, par_v)
    lo = par_v[...]
    width = jnp.full((_LANE,), 1 << _CSH, jnp.int32)

    @plsc.parallel_loop(0, _TBL // _LANE, unroll=8)
    def _zero(i):
        cnt_a[pl.ds(i * _LANE, _LANE)] = jnp.zeros((_LANE,), jnp.float32)
        cnt_b[pl.ds(i * _LANE, _LANE)] = jnp.zeros((_LANE,), jnp.float32)

    lane = lax.iota(jnp.int32, _LANE)
    ones = jnp.ones((_LANE,), jnp.float32)
    izero = jnp.zeros((_LANE,), jnp.int32)
    fzero = jnp.zeros((_LANE,), jnp.float32)
    shift = jnp.full((_LANE,), _FSH, jnp.int32)
    sixteen = jnp.full((_LANE,), _LANE, jnp.int32)

    b_idx = lax.shift_right_logical(wid, 1)
    r_base = (wid & 1) * 256
    bufs = (buf0, buf1)
    sems = (sem0, sem1)

    def _chunk(buf, acc):
        @plsc.parallel_loop(0, _CH // _LANE, step=2, unroll=_UNROLL // 2,
                            carry=acc)
        def _vreg(j, acc):
            for u, cnt_v in ((0, cnt_a), (1, cnt_b)):
                jj = j + u
                r = lax.shift_right_logical(jj, 5)
                cc = (jj & 31) * _LANE
                v = buf[r, pl.ds(cc, _LANE)]
                bits = lax.bitcast_convert_type(v, jnp.int32)
                rel = bits - lo
                inr = (rel >= izero) & (rel < width)
                fb = lax.shift_right_logical(rel, shift)
                fb = jnp.where(inr, fb, izero)
                idx = fb * sixteen + lane
                plsc.addupdate_scatter(cnt_v, [idx], ones, mask=inr)
                acc = acc + jnp.where(rel >= width, v, fzero)
            return acc

        return _vreg

    acc = fzero
    cur = pltpu.async_copy(
        neg_hbm.at[b_idx, pl.ds(r_base, _ROWS), :], buf0, sem0)
    for c in range(_NCH):
        nxt = None
        if c + 1 < _NCH:
            nxt = pltpu.async_copy(
                neg_hbm.at[b_idx, pl.ds(r_base + (c + 1) * _ROWS, _ROWS), :],
                bufs[(c + 1) % 2], sems[(c + 1) % 2])
        cur.wait()
        acc = _chunk(bufs[c % 2], acc)
        cur = nxt

    sab_v[...] = acc
    pltpu.sync_copy(sab_v, sab_hbm.at[wid])
    _tile_merge(cnt_a, cnt_b, shared, tmp_v, merged_v, cnt_hbm)


_CNT_OUT = jax.ShapeDtypeStruct((_NC, _TBL), jnp.float32)
_MESH = plsc.VectorSubcoreMesh(core_axis_name="c", subcore_axis_name="s")
_PARAMS = pltpu.CompilerParams(needs_layout_passes=False,
                               disable_bounds_checks=True)
_MERGE_SCRATCH = [
    pltpu.VMEM_SHARED((_NS, _TBL // 2), jnp.float32),
    pltpu.VMEM((_NS, _SL), jnp.float32),
    pltpu.VMEM((_SL,), jnp.float32),
]

_sc_coarse = pl.kernel(
    _coarse_body,
    out_type=[
        _CNT_OUT,
        jax.ShapeDtypeStruct((_B, 512, 512), jnp.float32),
        jax.ShapeDtypeStruct((_NW, 2, _LANE), jnp.float32),
    ],
    mesh=_MESH,
    compiler_params=_PARAMS,
    scratch_types=[pltpu.VMEM((_ROWS, 512), jnp.float32)] * 8
    + [pltpu.VMEM((2, _LANE), jnp.float32)]
    + [pltpu.VMEM((_TBL,), jnp.float32)] * 2
    + _MERGE_SCRATCH
    + [pltpu.SemaphoreType.DMA] * 8,
)

_sc_fine = pl.kernel(
    _fine_body,
    out_type=[_CNT_OUT, jax.ShapeDtypeStruct((_NW, _LANE), jnp.float32)],
    mesh=_MESH,
    compiler_params=_PARAMS,
    scratch_types=[pltpu.VMEM((_ROWS, 512), jnp.float32)] * 2
    + [pltpu.VMEM((_LANE,), jnp.int32), pltpu.VMEM((_LANE,), jnp.float32)]
    + [pltpu.VMEM((_TBL,), jnp.float32)] * 2
    + _MERGE_SCRATCH
    + [pltpu.SemaphoreType.DMA] * 2,
)


def _merge(tbl):
    return tbl.reshape(_NC, _NB, _LANE).sum(axis=(0, 2))


def _rev_cumsum(x):
    return jnp.cumsum(x[::-1])[::-1]


def kernel(pred, gt, mask):
    cnt_o, neg, stats = _sc_coarse(pred, gt, mask)

    pos_sum = stats[:, 0, :].sum()
    pos_cnt = jnp.floor(stats[:, 1, :].sum())
    neg_cnt = jnp.minimum(jnp.floor(float(_N) - stats[:, 1, :].sum()),
                          jnp.floor(pos_cnt * _NEG_RATIO))

    bins = jnp.arange(_NB, dtype=jnp.int32)

    cnt1 = _merge(cnt_o)
    h1 = _rev_cumsum(cnt1)
    b_star = jnp.max(jnp.where(h1 >= neg_cnt, bins, 0))
    ca = h1[b_star] - cnt1[b_star]

    lo = b_star << _CSH
    fcnt_o, sab_o = _sc_fine(neg, jnp.full((_LANE,), lo, jnp.int32))
    fcnt = _merge(fcnt_o)
    s_above = sab_o.sum()
    hf = _rev_cumsum(fcnt)
    f_star = jnp.max(jnp.where(ca + hf >= neg_cnt, bins, 0))
    c_abv = ca + hf[f_star] - fcnt[f_star]
    deficit = neg_cnt - c_abv

    centers = lax.bitcast_convert_type(
        lo + (bins << _FSH) + (1 << (_FSH - 1)), jnp.float32)
    wsum = _rev_cumsum(fcnt * centers)
    within = wsum[f_star] - fcnt[f_star] * centers[f_star]

    topk_sum = s_above + within + deficit * centers[f_star]
    negative_loss = topk_sum / neg_cnt
    positive_loss = pos_sum / pos_cnt
    total = positive_loss + negative_loss
    return (total, positive_loss, negative_loss)


# speedup vs baseline: 1.0402x; 1.0402x over previous
import jax
import jax.numpy as jnp
from jax import lax
from jax.experimental import pallas as pl
from jax.experimental.pallas import tpu as pltpu
from jax.experimental.pallas import tpu_sc as plsc

_NEG_RATIO = 3.0

_NC = 2
_NS = 16
_LANE = 16
_NW = _NC * _NS

_NB = 1024
_TBL = _NB * _LANE
_CSH = 21
_FSH = 11

_B = 16
_N = _B * 512 * 512
_CH = 8192
_ROWS = _CH // 512
_NCH = (_N // _NW) // _CH
_UNROLL = 8


def _coarse_body(pred_hbm, gt_hbm, mask_hbm, cnt_hbm, neg_hbm, stat_hbm,
                 bp0, bp1, bg0, bg1, bm0, bm1, bn0, bn1, stat_v,
                 cnt_a, cnt_b,
                 semp0, semp1, semg0, semg1, semm0, semm1, semw0, semw1):
    wid = lax.axis_index("s") * _NC + lax.axis_index("c")

    @plsc.parallel_loop(0, _TBL // _LANE, unroll=8)
    def _zero(i):
        cnt_a[pl.ds(i * _LANE, _LANE)] = jnp.zeros((_LANE,), jnp.float32)
        cnt_b[pl.ds(i * _LANE, _LANE)] = jnp.zeros((_LANE,), jnp.float32)

    lane = lax.iota(jnp.int32, _LANE)
    ones = jnp.ones((_LANE,), jnp.float32)
    fone = jnp.ones((_LANE,), jnp.float32)
    fzero = jnp.zeros((_LANE,), jnp.float32)
    shift = jnp.full((_LANE,), _CSH, jnp.int32)
    sixteen = jnp.full((_LANE,), _LANE, jnp.int32)

    b_idx = lax.shift_right_logical(wid, 1)
    r_base = (wid & 1) * 256
    bps = (bp0, bp1)
    bgs = (bg0, bg1)
    bms = (bm0, bm1)
    bns = (bn0, bn1)
    semws = (semw0, semw1)

    def _reads(c):
        rows = pl.ds(r_base + c * _ROWS, _ROWS)
        i = c % 2
        return (pltpu.async_copy(pred_hbm.at[b_idx, 0, rows, :], bps[i],
                                 (semp0, semp1)[i]),
                pltpu.async_copy(gt_hbm.at[b_idx, rows, :], bgs[i],
                                 (semg0, semg1)[i]),
                pltpu.async_copy(mask_hbm.at[b_idx, rows, :], bms[i],
                                 (semm0, semm1)[i]))

    def _chunk(i, carry):
        bp, bg, bm, bn = bps[i], bgs[i], bms[i], bns[i]

        @plsc.parallel_loop(0, _CH // _LANE, step=2, unroll=_UNROLL // 2,
                            carry=carry)
        def _vreg(j, carry):
            psum, pcnt = carry
            for u, cnt_v in ((0, cnt_a), (1, cnt_b)):
                jj = j + u
                r = lax.shift_right_logical(jj, 5)
                cc = (jj & 31) * _LANE
                p = bp[r, pl.ds(cc, _LANE)]
                g = bg[r, pl.ds(cc, _LANE)]
                m = bm[r, pl.ds(cc, _LANE)]
                loss = jnp.abs(p - g)
                lm = loss * m
                neg = loss - lm
                bn[r, pl.ds(cc, _LANE)] = neg
                psum = psum + lm
                pcnt = pcnt + m
                bits = lax.bitcast_convert_type(neg, jnp.int32)
                fb = lax.shift_right_logical(bits, shift)
                idx = fb * sixteen + lane
                plsc.addupdate_scatter(cnt_v, [idx], ones)
            return (psum, pcnt)

        return _vreg

    carry = (fzero, fzero)
    writes = [None, None]
    cur = _reads(0)
    for c in range(_NCH):
        nxt = _reads(c + 1) if c + 1 < _NCH else None
        for d in cur:
            d.wait()
        if writes[c % 2] is not None:
            writes[c % 2].wait()
        carry = _chunk(c % 2, carry)
        writes[c % 2] = pltpu.async_copy(
            bns[c % 2],
            neg_hbm.at[b_idx, pl.ds(r_base + c * _ROWS, _ROWS), :],
            semws[c % 2])
        cur = nxt

    for w in writes:
        if w is not None:
            w.wait()

    psum, pcnt = carry
    stat_v[0, :] = psum
    stat_v[1, :] = pcnt
    pltpu.sync_copy(cnt_a, cnt_hbm.at[wid, 0])
    pltpu.sync_copy(cnt_b, cnt_hbm.at[wid, 1])
    pltpu.sync_copy(stat_v, stat_hbm.at[wid])


def _fine_body(neg_hbm, par_hbm, cnt_hbm, sab_hbm,
               buf0, buf1, par_v, sab_v, cnt_a, cnt_b, sem0, sem1):
    wid = lax.axis_index("s") * _NC + lax.axis_index("c")

    pltpu.sync_copy(par_hbm, par_v)
    lo = par_v[...]
    width = jnp.full((_LANE,), 1 << _CSH, jnp.int32)

    @plsc.parallel_loop(0, _TBL // _LANE, unroll=8)
    def _zero(i):
        cnt_a[pl.ds(i * _LANE, _LANE)] = jnp.zeros((_LANE,), jnp.float32)
        cnt_b[pl.ds(i * _LANE, _LANE)] = jnp.zeros((_LANE,), jnp.float32)

    lane = lax.iota(jnp.int32, _LANE)
    ones = jnp.ones((_LANE,), jnp.float32)
    izero = jnp.zeros((_LANE,), jnp.int32)
    fzero = jnp.zeros((_LANE,), jnp.float32)
    shift = jnp.full((_LANE,), _FSH, jnp.int32)
    sixteen = jnp.full((_LANE,), _LANE, jnp.int32)

    b_idx = lax.shift_right_logical(wid, 1)
    r_base = (wid & 1) * 256
    bufs = (buf0, buf1)
    sems = (sem0, sem1)

    def _chunk(buf, acc):
        @plsc.parallel_loop(0, _CH // _LANE, step=2, unroll=_UNROLL // 2,
                            carry=acc)
        def _vreg(j, acc):
            for u, cnt_v in ((0, cnt_a), (1, cnt_b)):
                jj = j + u
                r = lax.shift_right_logical(jj, 5)
                cc = (jj & 31) * _LANE
                v = buf[r, pl.ds(cc, _LANE)]
                bits = lax.bitcast_convert_type(v, jnp.int32)
                rel = bits - lo
                inr = (rel >= izero) & (rel < width)
                fb = lax.shift_right_logical(rel, shift)
                fb = jnp.where(inr, fb, izero)
                idx = fb * sixteen + lane
                plsc.addupdate_scatter(cnt_v, [idx], ones, mask=inr)
                acc = acc + jnp.where(rel >= width, v, fzero)
            return acc

        return _vreg

    acc = fzero
    cur = pltpu.async_copy(
        neg_hbm.at[b_idx, pl.ds(r_base, _ROWS), :], buf0, sem0)
    for c in range(_NCH):
        nxt = None
        if c + 1 < _NCH:
            nxt = pltpu.async_copy(
                neg_hbm.at[b_idx, pl.ds(r_base + (c + 1) * _ROWS, _ROWS), :],
                bufs[(c + 1) % 2], sems[(c + 1) % 2])
        cur.wait()
        acc = _chunk(bufs[c % 2], acc)
        cur = nxt

    sab_v[...] = acc
    pltpu.sync_copy(cnt_a, cnt_hbm.at[wid, 0])
    pltpu.sync_copy(cnt_b, cnt_hbm.at[wid, 1])
    pltpu.sync_copy(sab_v, sab_hbm.at[wid])


_CNT_OUT = jax.ShapeDtypeStruct((_NW, 2, _TBL), jnp.float32)
_MESH = plsc.VectorSubcoreMesh(core_axis_name="c", subcore_axis_name="s")
_PARAMS = pltpu.CompilerParams(needs_layout_passes=False,
                               disable_bounds_checks=True)

_sc_coarse = pl.kernel(
    _coarse_body,
    out_type=[
        _CNT_OUT,
        jax.ShapeDtypeStruct((_B, 512, 512), jnp.float32),
        jax.ShapeDtypeStruct((_NW, 2, _LANE), jnp.float32),
    ],
    mesh=_MESH,
    compiler_params=_PARAMS,
    scratch_types=[pltpu.VMEM((_ROWS, 512), jnp.float32)] * 8
    + [pltpu.VMEM((2, _LANE), jnp.float32)]
    + [pltpu.VMEM((_TBL,), jnp.float32)] * 2
    + [pltpu.SemaphoreType.DMA] * 8,
)

_sc_fine = pl.kernel(
    _fine_body,
    out_type=[_CNT_OUT, jax.ShapeDtypeStruct((_NW, _LANE), jnp.float32)],
    mesh=_MESH,
    compiler_params=_PARAMS,
    scratch_types=[pltpu.VMEM((_ROWS, 512), jnp.float32)] * 2
    + [pltpu.VMEM((_LANE,), jnp.int32), pltpu.VMEM((_LANE,), jnp.float32)]
    + [pltpu.VMEM((_TBL,), jnp.float32)] * 2
    + [pltpu.SemaphoreType.DMA] * 2,
)


def _merge(tbl):
    return tbl.reshape(_NW * 2, _NB, _LANE).sum(axis=(0, 2))


def _rev_cumsum(x):
    return jnp.cumsum(x[::-1])[::-1]


def kernel(pred, gt, mask):
    cnt_o, neg, stats = _sc_coarse(pred, gt, mask)

    pos_sum = stats[:, 0, :].sum()
    pos_cnt = jnp.floor(stats[:, 1, :].sum())
    neg_cnt = jnp.minimum(jnp.floor(float(_N) - stats[:, 1, :].sum()),
                          jnp.floor(pos_cnt * _NEG_RATIO))

    bins = jnp.arange(_NB, dtype=jnp.int32)

    cnt1 = _merge(cnt_o)
    h1 = _rev_cumsum(cnt1)
    b_star = jnp.max(jnp.where(h1 >= neg_cnt, bins, 0))
    ca = h1[b_star] - cnt1[b_star]

    lo = b_star << _CSH
    fcnt_o, sab_o = _sc_fine(neg, jnp.full((_LANE,), lo, jnp.int32))
    fcnt = _merge(fcnt_o)
    s_above = sab_o.sum()
    hf = _rev_cumsum(fcnt)
    f_star = jnp.max(jnp.where(ca + hf >= neg_cnt, bins, 0))
    c_abv = ca + hf[f_star] - fcnt[f_star]
    deficit = neg_cnt - c_abv

    centers = lax.bitcast_convert_type(
        lo + (bins << _FSH) + (1 << (_FSH - 1)), jnp.float32)
    wsum = _rev_cumsum(fcnt * centers)
    within = wsum[f_star] - fcnt[f_star] * centers[f_star]

    topk_sum = s_above + within + deficit * centers[f_star]
    negative_loss = topk_sum / neg_cnt
    positive_loss = pos_sum / pos_cnt
    total = positive_loss + negative_loss
    return (total, positive_loss, negative_loss)
